# Initial kernel scaffold; baseline (speedup 1.0000x reference)
import jax
import jax.numpy as jnp
from jax.experimental import pallas as pl


def kernel(n_feat, edge_index, W_gcn, b_gcn, W_lin, b_lin):
    raise NotImplementedError("write your pallas kernel here")


# speedup vs baseline: 6.2480x; 6.2480x over previous
import jax
import jax.numpy as jnp
from jax import lax
from jax.experimental import pallas as pl
from jax.experimental.pallas import tpu as pltpu
from jax.experimental.pallas import tpu_sc as plsc

N = 10000
E = 320000
D_IN = 128
D_OUT = 40

NC = 2
NS = 16
NW = NC * NS
L = 16

CHUNK = 128
NCHUNKS = E // CHUNK

DEG_PAD = 10240
ROWS_PER_TILE = DEG_PAD // NS
ZROWS = 128

_MESH = plsc.VectorSubcoreMesh(
    core_axis_name="c", subcore_axis_name="s", num_cores=NC, num_subcores=NS
)


def _deg_body(edge_hbm, deg_hbm, idx_v, ones_v, zbuf_v, dacc, sem):
    cid = lax.axis_index("c")
    tid = lax.axis_index("s")

    def init_loop(i, carry):
        zbuf_v[pl.ds(i * L, L)] = jnp.zeros((L,), jnp.float32)
        return carry

    lax.fori_loop(0, (DEG_PAD // NS) // L, init_loop, 0)

    def ones_loop(i, carry):
        ones_v[pl.ds(i * L, L)] = jnp.ones((L,), jnp.float32)
        return carry

    lax.fori_loop(0, CHUNK // L, ones_loop, 0)

    pltpu.sync_copy(zbuf_v, dacc.at[pl.ds(tid * (DEG_PAD // NS), DEG_PAD // NS)])
    plsc.subcore_barrier()

    nch = jnp.where(tid < NCHUNKS % NS, NCHUNKS // NS + 1, NCHUNKS // NS)

    def body(j, carry):
        base = (tid + NS * j) * CHUNK
        pltpu.sync_copy(edge_hbm.at[cid, pl.ds(base, CHUNK)], idx_v)
        pltpu.sync_copy(ones_v, dacc.at[idx_v], add=True)
        return carry

    lax.fori_loop(0, nch, body, 0)
    plsc.subcore_barrier()

    s = tid * (DEG_PAD // NS)
    pltpu.sync_copy(
        dacc.at[pl.ds(s, DEG_PAD // NS)],
        deg_hbm.at[cid, pl.ds(s, DEG_PAD // NS)],
    )


_deg = pl.kernel(
    _deg_body,
    out_type=jax.ShapeDtypeStruct((NC, DEG_PAD), jnp.float32),
    mesh=_MESH,
    scratch_types=[
        pltpu.VMEM((CHUNK,), jnp.int32),
        pltpu.VMEM((CHUNK,), jnp.float32),
        pltpu.VMEM((DEG_PAD // NS,), jnp.float32),
        pltpu.VMEM_SHARED((DEG_PAD,), jnp.float32),
        pltpu.SemaphoreType.DMA,
    ],
)


def _agg_body(h_hbm, edge_hbm, accp_hbm, sidx_v, didx_v, rows_v, zb_v, acc, sem):
    cid = lax.axis_index("c")
    tid = lax.axis_index("s")
    wid = cid * NS + tid

    def zloop(i, carry):
        zb_v[i // (D_IN // L), pl.ds((i % (D_IN // L)) * L, L)] = jnp.zeros(
            (L,), jnp.float32
        )
        return carry

    lax.fori_loop(0, ZROWS * (D_IN // L), zloop, 0)

    def zcopy(k, carry):
        pltpu.sync_copy(zb_v, acc.at[pl.ds(tid * ROWS_PER_TILE + k * ZROWS, ZROWS)])
        return carry

    lax.fori_loop(0, ROWS_PER_TILE // ZROWS, zcopy, 0)
    plsc.subcore_barrier()

    nch = jnp.where(wid < NCHUNKS % NW, NCHUNKS // NW + 1, NCHUNKS // NW)

    def body(j, carry):
        base = (wid + NW * j) * CHUNK
        pltpu.sync_copy(edge_hbm.at[0, pl.ds(base, CHUNK)], sidx_v)
        pltpu.async_copy(h_hbm.at[sidx_v], rows_v, sem).wait()
        pltpu.sync_copy(edge_hbm.at[1, pl.ds(base, CHUNK)], didx_v)
        pltpu.sync_copy(rows_v, acc.at[didx_v], add=True)
        return carry

    lax.fori_loop(0, nch, body, 0)
    plsc.subcore_barrier()

    def wb(k, carry):
        r = tid * ROWS_PER_TILE + k * ZROWS
        pltpu.sync_copy(acc.at[pl.ds(r, ZROWS)], accp_hbm.at[cid, pl.ds(r, ZROWS)])
        return carry

    lax.fori_loop(0, ROWS_PER_TILE // ZROWS, wb, 0)


_agg = pl.kernel(
    _agg_body,
    out_type=jax.ShapeDtypeStruct((NC, DEG_PAD, D_IN), jnp.float32),
    mesh=_MESH,
    scratch_types=[
        pltpu.VMEM((CHUNK,), jnp.int32),
        pltpu.VMEM((CHUNK,), jnp.int32),
        pltpu.VMEM((CHUNK, D_IN), jnp.float32),
        pltpu.VMEM((ZROWS, D_IN), jnp.float32),
        pltpu.VMEM_SHARED((DEG_PAD, D_IN), jnp.float32),
        pltpu.SemaphoreType.DMA,
    ],
)


ROW_BLK = 1000


def _scale_body(x_ref, d_ref, o_ref):
    o_ref[...] = x_ref[...] * lax.rsqrt(jnp.maximum(d_ref[...], 1.0))


_scale = pl.pallas_call(
    _scale_body,
    out_shape=jax.ShapeDtypeStruct((N, D_IN), jnp.float32),
    grid=(N // ROW_BLK,),
    in_specs=[
        pl.BlockSpec((ROW_BLK, D_IN), lambda i: (i, 0)),
        pl.BlockSpec((ROW_BLK, 1), lambda i: (i, 0)),
    ],
    out_specs=pl.BlockSpec((ROW_BLK, D_IN), lambda i: (i, 0)),
)


def _head_body(a_ref, d_ref, w1_ref, b1_ref, w2_ref, b2_ref, o_ref):
    a = a_ref[0] + a_ref[1]
    a = a * lax.rsqrt(jnp.maximum(d_ref[...], 1.0))
    h2 = jnp.dot(a, w1_ref[...], preferred_element_type=jnp.float32) + b1_ref[...]
    h2 = jnp.maximum(h2, 0.0)
    o_ref[...] = (
        jnp.dot(h2, w2_ref[...], preferred_element_type=jnp.float32) + b2_ref[...]
    )


_head = pl.pallas_call(
    _head_body,
    out_shape=jax.ShapeDtypeStruct((N, D_OUT), jnp.float32),
    grid=(N // ROW_BLK,),
    in_specs=[
        pl.BlockSpec((NC, ROW_BLK, D_IN), lambda i: (0, i, 0)),
        pl.BlockSpec((ROW_BLK, 1), lambda i: (i, 0)),
        pl.BlockSpec((D_IN, D_IN), lambda i: (0, 0)),
        pl.BlockSpec((1, D_IN), lambda i: (0, 0)),
        pl.BlockSpec((D_IN, D_OUT), lambda i: (0, 0)),
        pl.BlockSpec((1, D_OUT), lambda i: (0, 0)),
    ],
    out_specs=pl.BlockSpec((ROW_BLK, D_OUT), lambda i: (i, 0)),
)


@jax.jit
def kernel(n_feat, edge_index, W_gcn, b_gcn, W_lin, b_lin):
    deg = _deg(edge_index)[:, :N]
    h = _scale(n_feat, deg[0].reshape(N, 1))
    accp = _agg(h, edge_index)[:, :N]
    out = _head(
        accp,
        deg[1].reshape(N, 1),
        W_gcn,
        b_gcn.reshape(1, D_IN),
        W_lin,
        b_lin.reshape(1, D_OUT),
    )
    return out
